# Initial kernel scaffold; baseline (speedup 1.0000x reference)
import functools

import jax
import jax.numpy as jnp
from jax import lax
from jax.experimental import pallas as pl
from jax.experimental.pallas import tpu as pltpu
from jax.experimental.pallas import tpu_sc as plsc

N = 819200
NC, NS, L = 2, 16, 16
NW = NC * NS
PER_W = N // NW
CHUNK = 1024
CR = CHUNK // 128
NCHUNK = PER_W // CHUNK
ROWS_PER_W = PER_W // 128
D = 16


def _sc_distance(xr, yr, table):
    mesh = plsc.VectorSubcoreMesh(
        core_axis_name="c", subcore_axis_name="s", num_cores=NC, num_subcores=NS
    )

    @functools.partial(
        pl.kernel,
        out_type=jax.ShapeDtypeStruct((N,), jnp.float32),
        mesh=mesh,
        scratch_types=[
            pltpu.VMEM((CR, 128), jnp.int32),
            pltpu.VMEM((CR, 128), jnp.int32),
            pltpu.VMEM((CHUNK, D), jnp.float32),
            pltpu.VMEM((CHUNK, D), jnp.float32),
            pltpu.VMEM((CHUNK,), jnp.float32),
            pltpu.SemaphoreType.DMA,
        ],
    )
    def k(x_hbm, y_hbm, tbl_hbm, out_hbm, idx_x, idx_y, rows_x, rows_y, dout, sem):
        wid = lax.axis_index("s") * NC + lax.axis_index("c")

        def chunk_body(c, carry):
            row0 = wid * ROWS_PER_W + c * CR
            e0 = wid * PER_W + c * CHUNK
            pltpu.sync_copy(x_hbm.at[pl.ds(row0, CR)], idx_x)
            pltpu.sync_copy(y_hbm.at[pl.ds(row0, CR)], idx_y)
            cps = []
            for j in range(CR):
                cps.append(pltpu.async_copy(
                    tbl_hbm.at[idx_x.at[j]], rows_x.at[pl.ds(j * 128, 128)], sem))
                cps.append(pltpu.async_copy(
                    tbl_hbm.at[idx_y.at[j]], rows_y.at[pl.ds(j * 128, 128)], sem))
            for cp in cps:
                cp.wait()

            def group_body(g, carry2):
                ge = g * L
                eidx = ge + lax.iota(jnp.int32, L)
                uu = jnp.zeros((L,), jnp.float32)
                vv = jnp.zeros((L,), jnp.float32)
                ww = jnp.zeros((L,), jnp.float32)
                for dd in range(D):
                    dvec = jnp.full((L,), dd, jnp.int32)
                    cu = plsc.load_gather(rows_x, [eidx, dvec])
                    cv = plsc.load_gather(rows_y, [eidx, dvec])
                    uu = uu + cu * cu
                    vv = vv + cv * cv
                    w = cu - cv
                    ww = ww + w * w
                den = (1.0 - uu) * (1.0 - vv)
                dout[pl.ds(ge, L)] = 1.0 + 2.0 * ww / den
                return carry2

            lax.fori_loop(0, CHUNK // L, group_body, 0)
            pltpu.sync_copy(dout, out_hbm.at[pl.ds(e0, CHUNK)])
            return carry

        lax.fori_loop(0, NCHUNK, chunk_body, 0)

    return k(xr, yr, table)


def _acosh_body(d_ref, o_ref):
    o_ref[...] = -jnp.arccosh(d_ref[...])


def _tc_neg_acosh(d):
    d2 = d.reshape(N // 128, 128)
    out = pl.pallas_call(
        _acosh_body,
        out_shape=jax.ShapeDtypeStruct((N // 128, 128), jnp.float32),
    )(d2)
    return out


def kernel(x, y, embedding):
    xr = x.reshape(N // 128, 128)
    yr = y.reshape(N // 128, 128)
    d = _sc_distance(xr, yr, embedding)
    return _tc_neg_acosh(d).reshape(x.shape)


# speedup vs baseline: 1.9819x; 1.9819x over previous
import functools

import jax
import jax.numpy as jnp
from jax import lax
from jax.experimental import pallas as pl
from jax.experimental.pallas import tpu as pltpu
from jax.experimental.pallas import tpu_sc as plsc

N = 819200
NC, NS, L = 2, 16, 16
NW = NC * NS
PER_W = N // NW
CHUNK = 1024
CR = CHUNK // 128
NCHUNK = PER_W // CHUNK
ROWS_PER_W = PER_W // 128
D = 16


def _sc_distance(xr, yr, table):
    mesh = plsc.VectorSubcoreMesh(
        core_axis_name="c", subcore_axis_name="s", num_cores=NC, num_subcores=NS
    )

    @functools.partial(
        pl.kernel,
        out_type=jax.ShapeDtypeStruct((N,), jnp.float32),
        mesh=mesh,
        compiler_params=pltpu.CompilerParams(
            needs_layout_passes=False, use_tc_tiling_on_sc=False),
        scratch_types=[
            pltpu.VMEM((CR, 128), jnp.int32),
            pltpu.VMEM((CR, 128), jnp.int32),
            pltpu.VMEM((CHUNK, D), jnp.float32),
            pltpu.VMEM((CHUNK, D), jnp.float32),
            pltpu.VMEM((CHUNK,), jnp.float32),
            pltpu.SemaphoreType.DMA,
        ],
    )
    def k(x_hbm, y_hbm, tbl_hbm, out_hbm, idx_x, idx_y, rows_x, rows_y, dout, sem):
        wid = lax.axis_index("s") * NC + lax.axis_index("c")

        def chunk_body(c, carry):
            row0 = wid * ROWS_PER_W + c * CR
            e0 = wid * PER_W + c * CHUNK
            pltpu.sync_copy(x_hbm.at[pl.ds(row0, CR)], idx_x)
            pltpu.sync_copy(y_hbm.at[pl.ds(row0, CR)], idx_y)
            cps = []
            for j in range(CR):
                cps.append(pltpu.async_copy(
                    tbl_hbm.at[idx_x.at[j]], rows_x.at[pl.ds(j * 128, 128)], sem))
                cps.append(pltpu.async_copy(
                    tbl_hbm.at[idx_y.at[j]], rows_y.at[pl.ds(j * 128, 128)], sem))
            for cp in cps:
                cp.wait()

            def group_body(g, carry2):
                ge = g * L
                eidx = ge + lax.iota(jnp.int32, L)
                uu = jnp.zeros((L,), jnp.float32)
                vv = jnp.zeros((L,), jnp.float32)
                ww = jnp.zeros((L,), jnp.float32)
                for dd in range(D):
                    dvec = jnp.full((L,), dd, jnp.int32)
                    cu = plsc.load_gather(rows_x, [eidx, dvec])
                    cv = plsc.load_gather(rows_y, [eidx, dvec])
                    uu = uu + cu * cu
                    vv = vv + cv * cv
                    w = cu - cv
                    ww = ww + w * w
                den = (1.0 - uu) * (1.0 - vv)
                dout[pl.ds(ge, L)] = 1.0 + 2.0 * ww / den
                return carry2

            lax.fori_loop(0, CHUNK // L, group_body, 0)
            pltpu.sync_copy(dout, out_hbm.at[pl.ds(e0, CHUNK)])
            return carry

        lax.fori_loop(0, NCHUNK, chunk_body, 0)

    return k(xr, yr, table)


def _acosh_body(d_ref, o_ref):
    d = d_ref[...]
    o_ref[...] = -jnp.log(d + jnp.sqrt((d - 1.0) * (d + 1.0)))


def _tc_neg_acosh(d):
    d2 = d.reshape(N // 128, 128)
    out = pl.pallas_call(
        _acosh_body,
        out_shape=jax.ShapeDtypeStruct((N // 128, 128), jnp.float32),
    )(d2)
    return out


def kernel(x, y, embedding):
    xr = x.reshape(N // 128, 128)
    yr = y.reshape(N // 128, 128)
    d = _sc_distance(xr, yr, embedding)
    return _tc_neg_acosh(d).reshape(x.shape)


# speedup vs baseline: 2.1310x; 1.0752x over previous
import functools

import jax
import jax.numpy as jnp
from jax import lax
from jax.experimental import pallas as pl
from jax.experimental.pallas import tpu as pltpu
from jax.experimental.pallas import tpu_sc as plsc

N = 819200
NC, NS, L = 2, 16, 16
NW = NC * NS
PER_W = N // NW
CHUNK = 1280
CR = CHUNK // 128
NCHUNK = PER_W // CHUNK
D = 16


def _sc_distance(xf, yf, table):
    mesh = plsc.VectorSubcoreMesh(
        core_axis_name="c", subcore_axis_name="s", num_cores=NC, num_subcores=NS
    )

    @functools.partial(
        pl.kernel,
        out_type=jax.ShapeDtypeStruct((N,), jnp.float32),
        mesh=mesh,
        compiler_params=pltpu.CompilerParams(
            needs_layout_passes=False, use_tc_tiling_on_sc=False),
        scratch_types=[
            pltpu.VMEM((CHUNK,), jnp.int32),
            pltpu.VMEM((CHUNK,), jnp.int32),
            pltpu.VMEM((CHUNK,), jnp.int32),
            pltpu.VMEM((CHUNK,), jnp.int32),
            pltpu.VMEM((CHUNK, D), jnp.float32),
            pltpu.VMEM((CHUNK, D), jnp.float32),
            pltpu.VMEM((CHUNK, D), jnp.float32),
            pltpu.VMEM((CHUNK, D), jnp.float32),
            pltpu.VMEM((CHUNK,), jnp.float32),
            pltpu.VMEM((CHUNK,), jnp.float32),
            pltpu.SemaphoreType.DMA,
            pltpu.SemaphoreType.DMA,
        ],
    )
    def k(x_hbm, y_hbm, tbl_hbm, out_hbm,
          ix0, iy0, ix1, iy1, rx0, ry0, rx1, ry1, do0, do1, sem0, sem1):
        wid = lax.axis_index("s") * NC + lax.axis_index("c")

        def stage_fire(c, ix, iy, rx, ry, sem):
            e0 = wid * PER_W + c * CHUNK
            pltpu.sync_copy(x_hbm.at[pl.ds(e0, CHUNK)], ix)
            pltpu.sync_copy(y_hbm.at[pl.ds(e0, CHUNK)], iy)
            for j in range(CR):
                pltpu.async_copy(
                    tbl_hbm.at[ix.at[pl.ds(j * 128, 128)]],
                    rx.at[pl.ds(j * 128, 128)], sem)
                pltpu.async_copy(
                    tbl_hbm.at[iy.at[pl.ds(j * 128, 128)]],
                    ry.at[pl.ds(j * 128, 128)], sem)

        def drain(rx, ry, sem):
            pltpu.make_async_copy(tbl_hbm.at[pl.ds(0, CHUNK)], rx, sem).wait()
            pltpu.make_async_copy(tbl_hbm.at[pl.ds(0, CHUNK)], ry, sem).wait()

        def compute(c, rx, ry, do):
            e0 = wid * PER_W + c * CHUNK

            def group_body(g, carry2):
                ge = g * L
                eidx = ge + lax.iota(jnp.int32, L)
                uu = jnp.zeros((L,), jnp.float32)
                vv = jnp.zeros((L,), jnp.float32)
                uv = jnp.zeros((L,), jnp.float32)
                for dd in range(D):
                    dvec = jnp.full((L,), dd, jnp.int32)
                    cu = plsc.load_gather(rx, [eidx, dvec])
                    cv = plsc.load_gather(ry, [eidx, dvec])
                    uu = uu + cu * cu
                    vv = vv + cv * cv
                    uv = uv + cu * cv
                ww = jnp.maximum(uu + vv - 2.0 * uv, 0.0)
                den = (1.0 - uu) * (1.0 - vv)
                do[pl.ds(ge, L)] = 1.0 + 2.0 * ww / den
                return carry2

            lax.fori_loop(0, CHUNK // L, group_body, 0)
            pltpu.sync_copy(do, out_hbm.at[pl.ds(e0, CHUNK)])

        with jax.named_scope("prologue_fire"):
            stage_fire(0, ix0, iy0, rx0, ry0, sem0)

        def body(i, carry):
            c0 = 2 * i
            stage_fire(c0 + 1, ix1, iy1, rx1, ry1, sem1)
            with jax.named_scope("drain0"):
                drain(rx0, ry0, sem0)
            with jax.named_scope("compute0"):
                compute(c0, rx0, ry0, do0)
            stage_fire(c0 + 2, ix0, iy0, rx0, ry0, sem0)
            with jax.named_scope("drain1"):
                drain(rx1, ry1, sem1)
            with jax.named_scope("compute1"):
                compute(c0 + 1, rx1, ry1, do1)
            return carry

        lax.fori_loop(0, NCHUNK // 2 - 1, body, 0)

        c0 = NCHUNK - 2
        stage_fire(c0 + 1, ix1, iy1, rx1, ry1, sem1)
        drain(rx0, ry0, sem0)
        compute(c0, rx0, ry0, do0)
        drain(rx1, ry1, sem1)
        compute(c0 + 1, rx1, ry1, do1)

    return k(xf, yf, table)


def _acosh_body(d_ref, o_ref):
    d = d_ref[...]
    o_ref[...] = -jnp.log(d + jnp.sqrt((d - 1.0) * (d + 1.0)))


def _tc_neg_acosh(d):
    d2 = d.reshape(N // 128, 128)
    out = pl.pallas_call(
        _acosh_body,
        out_shape=jax.ShapeDtypeStruct((N // 128, 128), jnp.float32),
    )(d2)
    return out


def kernel(x, y, embedding):
    xf = x.reshape(N)
    yf = y.reshape(N)
    d = _sc_distance(xf, yf, embedding)
    return _tc_neg_acosh(d).reshape(x.shape)
